# baseline (speedup 1.0000x reference)
import math

import jax
import jax.numpy as jnp
from jax.experimental import pallas as pl

N = 10000
E = 320000
H = 8
C = 16
TD = 64
HC = H * C


def _copy_body(x_ref, o_ref):
    o_ref[...] = x_ref[...]


def _time_encode(ts, freq):
    tn = jnp.log1p(ts + 1e-6)
    ang = tn * freq[None, :]
    te = jnp.zeros((ts.shape[0], TD), dtype=ts.dtype)
    te = te.at[:, 0::2].set(jnp.sin(ang))
    te = te.at[:, 1::2].set(jnp.cos(ang))
    return te


def _gat_layer(x, edge_index, edge_attr, timestamps, Wq, Wk, Wv, freq, tb, Wep, bep):
    n = x.shape[0]
    q = (x @ Wq.T).reshape(n, H, C)
    k = (x @ Wk.T).reshape(n, H, C)
    v = (x @ Wv.T).reshape(n, H, C)
    src = edge_index[0]
    dst = edge_index[1]
    attn = (q[src] * k[dst]).sum(-1) / math.sqrt(C)
    ts = timestamps.reshape(-1, 1)
    te = _time_encode(ts, freq)
    mx = ts.max()
    mn = ts.min()
    time_norm = jnp.where(mx > mn, (ts - mn) / (mx - mn + 1e-6), jnp.ones_like(ts))
    attn = attn + tb[None, :] * time_norm
    edge_time = jnp.concatenate([edge_attr, te], axis=-1)
    attn = attn + edge_time @ Wep.T + bep
    attn = jax.nn.leaky_relu(attn, 0.2)
    aexp = jnp.exp(attn)
    asum = jax.ops.segment_sum(aexp, dst, num_segments=n)
    anorm = aexp / (asum[dst] + 1e-8)
    wv = anorm[..., None] * v[src]
    out = jax.ops.segment_sum(wv, dst, num_segments=n)
    return out.reshape(n, HC)


def kernel(x, edge_index, edge_attr, timestamps, Wq1, Wk1, Wv1, freq1, tb1, Wep1, bep1, Wq2, Wk2, Wv2, freq2, tb2, Wep2, bep2, Wout, bout):
    with jax.default_matmul_precision("highest"):
        return _kernel_impl(x, edge_index, edge_attr, timestamps, Wq1, Wk1, Wv1, freq1, tb1, Wep1, bep1, Wq2, Wk2, Wv2, freq2, tb2, Wep2, bep2, Wout, bout)


def _kernel_impl(x, edge_index, edge_attr, timestamps, Wq1, Wk1, Wv1, freq1, tb1, Wep1, bep1, Wq2, Wk2, Wv2, freq2, tb2, Wep2, bep2, Wout, bout):
    h = _gat_layer(x, edge_index, edge_attr, timestamps, Wq1, Wk1, Wv1, freq1, tb1, Wep1, bep1)
    h = jax.nn.elu(h)
    h = _gat_layer(h, edge_index, edge_attr, timestamps, Wq2, Wk2, Wv2, freq2, tb2, Wep2, bep2)
    h = jax.nn.elu(h)
    return h @ Wout.T + bout


# speedup vs baseline: 28.5816x; 28.5816x over previous
import functools
import math

import jax
import jax.numpy as jnp
from jax import lax
from jax.experimental import pallas as pl
from jax.experimental.pallas import tpu as pltpu
from jax.experimental.pallas import tpu_sc as plsc

N = 10000
E = 320000
DIN = 128
H = 8
C = 16
TD = 64
ED = 16
HC = H * C

NC = 2
NS = 16
NW = NC * NS
EPW = E // NW
BLK = 64
NBLK = 158
AC = 144
CH = 624
TAIL = N - NS * CH

F32 = jnp.float32
I32 = jnp.int32
_PREC = jax.lax.Precision.HIGHEST


def _prep_body(ts_ref, tlog_ref, tnorm_ref):
    t = ts_ref[...]
    mn = jnp.min(t)
    mx = jnp.max(t)
    tlog_ref[...] = jnp.log1p(t + 1e-6)
    tnorm_ref[...] = jnp.where(mx > mn, (t - mn) / (mx - mn + 1e-6),
                               jnp.ones_like(t))


def _prep(ts2d):
    return pl.pallas_call(
        _prep_body,
        out_shape=[jax.ShapeDtypeStruct((E // 128, 128), F32)] * 2,
    )(ts2d)


def _base_body(tlog_ref, tnorm_ref, ea_ref, fcat_ref, ph_ref, tb_ref, bep_ref,
               wea_ref, wte_ref, out_ref):
    ang = tlog_ref[...] * fcat_ref[...] + ph_ref[...]
    te = jnp.sin(ang)
    acc = jnp.dot(te, wte_ref[...], precision=_PREC)
    acc += jnp.dot(ea_ref[...], wea_ref[...], precision=_PREC)
    out_ref[...] = acc + tnorm_ref[...] * tb_ref[...] + bep_ref[...]


def _edge_bases(tlog_col, tnorm_col, ea, fcat, ph, tb, bep, wea, wte):
    nb = 80
    eb = E // nb
    full = lambda s: pl.BlockSpec(s, lambda i: (0, 0))
    return pl.pallas_call(
        _base_body,
        grid=(nb,),
        in_specs=[
            pl.BlockSpec((eb, 1), lambda i: (i, 0)),
            pl.BlockSpec((eb, 1), lambda i: (i, 0)),
            pl.BlockSpec((eb, ED), lambda i: (i, 0)),
            full((1, 128)), full((1, 128)), full((1, 16)), full((1, 16)),
            full((ED, 16)), full((128, 16)),
        ],
        out_specs=pl.BlockSpec((eb, 16), lambda i: (i, 0)),
        out_shape=jax.ShapeDtypeStruct((E, 16), F32),
    )(tlog_col, tnorm_col, ea, fcat, ph, tb, bep, wea, wte)


def _qkv_body(x_ref, w_ref, qv_ref, k_ref):
    r = jnp.dot(x_ref[...], w_ref[...], precision=_PREC)
    qv_ref[...] = r[:, :2 * HC]
    k_ref[...] = r[:, 2 * HC:]


def _qkv(x, wqvk):
    nb = 5
    rb = N // nb
    return pl.pallas_call(
        _qkv_body,
        grid=(nb,),
        in_specs=[
            pl.BlockSpec((rb, DIN), lambda i: (i, 0)),
            pl.BlockSpec((DIN, 3 * HC), lambda i: (0, 0)),
        ],
        out_specs=[
            pl.BlockSpec((rb, 2 * HC), lambda i: (i, 0)),
            pl.BlockSpec((rb, HC), lambda i: (i, 0)),
        ],
        out_shape=[
            jax.ShapeDtypeStruct((N, 2 * HC), F32),
            jax.ShapeDtypeStruct((N, HC), F32),
        ],
    )(x, wqvk)


def _norm_elu(acc_ref):
    m = acc_ref[0] + acc_ref[1]
    a = m[:, HC:HC + 16]
    rec = 1.0 / (a + 1e-8)
    hsel = lax.broadcasted_iota(I32, (16, HC), 0)
    jsel = lax.broadcasted_iota(I32, (16, HC), 1) // C
    sel = (hsel == jsel).astype(F32)
    rexp = jnp.dot(rec, sel, precision=_PREC)
    hid = m[:, :HC] * rexp
    return jnp.where(hid > 0, hid, jnp.exp(hid) - 1.0)


def _mid_body(acc_ref, w_ref, qv_ref, k_ref):
    eh = _norm_elu(acc_ref)
    r = jnp.dot(eh, w_ref[...], precision=_PREC)
    qv_ref[...] = r[:, :2 * HC]
    k_ref[...] = r[:, 2 * HC:]


def _mid(acc, wqvk):
    nb = 5
    rb = N // nb
    return pl.pallas_call(
        _mid_body,
        grid=(nb,),
        in_specs=[
            pl.BlockSpec((2, rb, AC), lambda i: (0, i, 0)),
            pl.BlockSpec((DIN, 3 * HC), lambda i: (0, 0)),
        ],
        out_specs=[
            pl.BlockSpec((rb, 2 * HC), lambda i: (i, 0)),
            pl.BlockSpec((rb, HC), lambda i: (i, 0)),
        ],
        out_shape=[
            jax.ShapeDtypeStruct((N, 2 * HC), F32),
            jax.ShapeDtypeStruct((N, HC), F32),
        ],
    )(acc, wqvk)


def _final_body(acc_ref, w_ref, b_ref, out_ref):
    eh = _norm_elu(acc_ref)
    out_ref[...] = jnp.dot(eh, w_ref[...], precision=_PREC) + b_ref[...]


def _final(acc, wout_t, bout2d):
    nb = 5
    rb = N // nb
    return pl.pallas_call(
        _final_body,
        grid=(nb,),
        in_specs=[
            pl.BlockSpec((2, rb, AC), lambda i: (0, i, 0)),
            pl.BlockSpec((DIN, 2), lambda i: (0, 0)),
            pl.BlockSpec((1, 2), lambda i: (0, 0)),
        ],
        out_specs=pl.BlockSpec((rb, 2), lambda i: (i, 0)),
        out_shape=jax.ShapeDtypeStruct((N, 2), F32),
    )(acc, wout_t, bout2d)


def _sc_layer(qv, k, base, edge_index, zeros, layer):
    mesh = plsc.VectorSubcoreMesh(core_axis_name="c", subcore_axis_name="s",
                                  num_cores=NC, num_subcores=NS)
    col0 = 8 * layer
    isqrtc = 1.0 / math.sqrt(C)

    @functools.partial(
        pl.kernel,
        out_type=jax.ShapeDtypeStruct((NC, N, AC), F32),
        mesh=mesh,
        compiler_params=pltpu.CompilerParams(use_tc_tiling_on_sc=False,
                                             needs_layout_passes=False),
        scratch_types=[
            pltpu.VMEM((2, 2, BLK), I32),
            pltpu.VMEM((2, BLK, 17), F32),
            pltpu.VMEM((BLK, 2 * HC), F32),
            pltpu.VMEM((BLK, HC), F32),
            pltpu.VMEM((BLK, AC), F32),
            pltpu.VMEM_SHARED((N, AC), F32),
            pltpu.SemaphoreType.DMA,
            pltpu.SemaphoreType.DMA,
            pltpu.SemaphoreType.DMA,
            pltpu.SemaphoreType.DMA,
        ],
    )
    def body(qv_hbm, k_hbm, base_hbm, ei_hbm, zero_hbm,
             out_acc, sd_v, base_v, qv_rows, k_rows, out_buf, acc,
             sem_q, sem_k, sem_sd, sem_b):
        core = lax.axis_index("c")
        sub = lax.axis_index("s")
        wid = sub * NC + core
        r0 = sub * CH

        pltpu.sync_copy(zero_hbm, acc.at[pl.ds(r0, CH), :])

        @pl.when(sub == NS - 1)
        def _zero_tail():
            pltpu.sync_copy(zero_hbm.at[pl.ds(0, TAIL), :],
                            acc.at[pl.ds(NS * CH, TAIL), :])

        def zrow(i, _):
            out_buf[i, pl.ds(HC, 16)] = jnp.zeros((16,), F32)
            return 0
        lax.fori_loop(0, BLK, zrow, 0)

        plsc.subcore_barrier()

        lanes = lax.iota(I32, 16)
        ebase = wid * EPW

        def eoff(b):
            return ebase + jnp.minimum(b * BLK, EPW - BLK)

        def prefetch(b, buf):
            e0 = eoff(b)
            pltpu.async_copy(ei_hbm.at[:, pl.ds(e0, BLK)], sd_v.at[buf], sem_sd)
            pltpu.async_copy(base_hbm.at[pl.ds(e0, BLK), :],
                             base_v.at[buf, :, pl.ds(0, 16)], sem_b)

        def drain_prefetch(buf):
            pltpu.make_async_copy(ei_hbm.at[:, pl.ds(0, BLK)], sd_v.at[buf],
                                  sem_sd).wait()
            pltpu.make_async_copy(base_hbm.at[pl.ds(0, BLK), :],
                                  base_v.at[buf, :, pl.ds(0, 16)], sem_b).wait()

        def process(b, cur, nxt):
            drain_prefetch(cur)
            cp1 = pltpu.async_copy(qv_hbm.at[sd_v.at[cur, 0]], qv_rows, sem_q)
            cp2 = pltpu.async_copy(k_hbm.at[sd_v.at[cur, 1]], k_rows, sem_k)
            prefetch(b + 1, nxt)
            cp1.wait()
            cp2.wait()
            thresh = jnp.where(
                b == NBLK - 2, NBLK * BLK - EPW - BLK,
                jnp.where(b == NBLK - 1, BLK, 0))

            def grp_body(g):
                rows = g * 16 + lanes

                def dot_step(c, ds):
                    rc = (lanes + c) & 15
                    out = []
                    for h in range(H):
                        cc = h * C + rc
                        qc = plsc.load_gather(qv_rows, [rows, cc])
                        kc = plsc.load_gather(k_rows, [rows, cc])
                        out.append(ds[h] + qc * kc)
                    return tuple(out)

                ds = lax.fori_loop(
                    0, C, dot_step,
                    tuple(jnp.zeros((16,), F32) for _ in range(H)))
                avs = []
                for h in range(H):
                    a = ds[h] * isqrtc
                    a = a + plsc.load_gather(
                        base_v.at[cur], [rows, jnp.full((16,), col0 + h, I32)])
                    a = jnp.where(a > 0, a, a * 0.2)
                    a = jnp.exp(a)
                    a = jnp.where(rows >= thresh, a, 0.0)
                    plsc.store_scatter(out_buf, [rows, jnp.full((16,), HC + h, I32)], a)
                    avs.append(a)

                def av_step(c, carry):
                    rc = (lanes + c) & 15
                    for h in range(H):
                        col = h * C + rc
                        vc = plsc.load_gather(qv_rows, [rows, HC + col])
                        plsc.store_scatter(out_buf, [rows, col], avs[h] * vc)
                    return carry

                lax.fori_loop(0, C, av_step, 0)

            plsc.parallel_loop(0, BLK // 16, step=1, unroll=1)(grp_body)
            pltpu.sync_copy(out_buf, acc.at[sd_v.at[cur, 1]], add=True)

        prefetch(0, 0)

        def pair_body(p, _):
            process(2 * p, 0, 1)
            process(2 * p + 1, 1, 0)
            return 0

        lax.fori_loop(0, NBLK // 2, pair_body, 0)
        drain_prefetch(0)

        plsc.subcore_barrier()

        pltpu.sync_copy(acc.at[pl.ds(r0, CH), :], out_acc.at[core, pl.ds(r0, CH), :])

        @pl.when(sub == NS - 1)
        def _write_tail():
            pltpu.sync_copy(acc.at[pl.ds(NS * CH, TAIL), :],
                            out_acc.at[core, pl.ds(NS * CH, TAIL), :])

    return body(qv, k, base, edge_index, zeros)


def _wep_split(wep):
    wea = wep[:, :ED].T
    wsin = wep[:, ED::2].T
    wcos = wep[:, ED + 1::2].T
    return wea, wsin, wcos


def kernel(x, edge_index, edge_attr, timestamps, Wq1, Wk1, Wv1, freq1, tb1,
           Wep1, bep1, Wq2, Wk2, Wv2, freq2, tb2, Wep2, bep2, Wout, bout):
    zeros = jnp.zeros((CH, AC), F32)

    tlog2, tnorm2 = _prep(timestamps.reshape(E // 128, 128))

    wea1, wsin1, wcos1 = _wep_split(Wep1)
    wea2, wsin2, wcos2 = _wep_split(Wep2)
    z32_8 = jnp.zeros((32, 8), F32)
    wte = jnp.concatenate([
        jnp.concatenate([wsin1, z32_8], axis=1),
        jnp.concatenate([wcos1, z32_8], axis=1),
        jnp.concatenate([z32_8, wsin2], axis=1),
        jnp.concatenate([z32_8, wcos2], axis=1),
    ], axis=0)
    fcat = jnp.concatenate([freq1, freq1, freq2, freq2]).reshape(1, 128)
    half_pi = jnp.float32(math.pi / 2)
    ph = jnp.concatenate([
        jnp.zeros((32,), F32), jnp.full((32,), half_pi, F32),
        jnp.zeros((32,), F32), jnp.full((32,), half_pi, F32),
    ]).reshape(1, 128)
    base = _edge_bases(
        tlog2.reshape(E, 1), tnorm2.reshape(E, 1), edge_attr, fcat, ph,
        jnp.concatenate([tb1, tb2]).reshape(1, 16),
        jnp.concatenate([bep1, bep2]).reshape(1, 16),
        jnp.concatenate([wea1, wea2], axis=1),
        wte,
    )

    wqvk1 = jnp.concatenate([Wq1.T, Wv1.T, Wk1.T], axis=1)
    wqvk2 = jnp.concatenate([Wq2.T, Wv2.T, Wk2.T], axis=1)

    qv1, k1 = _qkv(x, wqvk1)
    acc1 = _sc_layer(qv1, k1, base, edge_index, zeros, 0)
    qv2, k2 = _mid(acc1, wqvk2)
    acc2 = _sc_layer(qv2, k2, base, edge_index, zeros, 1)
    return _final(acc2, Wout.T, bout.reshape(1, 2))
